# baseline (speedup 1.0000x reference)
import jax
import jax.numpy as jnp
from jax.experimental import pallas as pl

ROWS = 4096


def _fused_kernel(x_ref, feat_ref, w1_ref, w2_ref, pos2_ref, out_ref):
    x = x_ref[:]
    n_cont = x.shape[1] - 3
    r = x.shape[0]
    d6 = w1_ref.shape[1]
    feat_ref[:, :n_cont] = x[:, :n_cont].astype(jnp.bfloat16)
    idx = x[:, n_cont:].astype(jnp.int32)
    j = jax.lax.broadcasted_iota(jnp.int32, (r, 128), 1)
    oh = (j == idx[:, 1:2]) | (j == idx[:, 0:1] + 15) | (j == idx[:, 2:3] + 79)
    out_ref[:, :d6] = jnp.dot(feat_ref[:], w1_ref[:],
                              preferred_element_type=jnp.float32)
    mm2 = jnp.dot(oh.astype(jnp.bfloat16), w2_ref[:],
                  preferred_element_type=jnp.float32)
    out_ref[:, d6:] = mm2 + pos2_ref[:]


def kernel(x, W, b, table_dd, table_plate, table_mag, table_pos):
    B, S, F = x.shape
    n_cont = F - 3
    d6 = W.shape[1]
    d9 = table_dd.shape[1]
    d_model = d6 + 3 * d9
    N = B * S

    n_dd = table_dd.shape[0]
    n_plate = table_plate.shape[0]
    n_mag = table_mag.shape[0]

    W1e = jnp.zeros((128, d6), jnp.float32)
    W1e = W1e.at[:n_cont, :].set(W)
    W1e = W1e.at[n_cont:n_cont + S, :].set(table_pos[:S, :d6])
    W1e = W1e.at[n_cont + S, :].set(b)
    W1e = W1e.astype(jnp.bfloat16)

    W2 = jnp.zeros((128, 3 * d9), jnp.float32)
    W2 = W2.at[:n_dd, :d9].set(table_dd)
    W2 = W2.at[n_dd:n_dd + n_plate, d9:2 * d9].set(table_plate)
    W2 = W2.at[n_dd + n_plate:n_dd + n_plate + n_mag, 2 * d9:].set(table_mag)
    W2 = W2.astype(jnp.bfloat16)

    rmod = jnp.arange(ROWS, dtype=jnp.int32) % S
    feat0 = jnp.zeros((ROWS, 128), jnp.bfloat16)
    feat0 = feat0.at[:, n_cont:n_cont + S].set(
        jax.nn.one_hot(rmod, S, dtype=jnp.bfloat16))
    feat0 = feat0.at[:, n_cont + S].set(jnp.bfloat16(1))

    pos2 = jnp.tile(table_pos[:S, d6:], (ROWS // S, 1))

    x2 = x.reshape(N, F)
    out = pl.pallas_call(
        _fused_kernel,
        grid=(N // ROWS,),
        in_specs=[
            pl.BlockSpec((ROWS, F), lambda i: (i, 0)),
            pl.BlockSpec((ROWS, 128), lambda i: (0, 0)),
            pl.BlockSpec((128, d6), lambda i: (0, 0)),
            pl.BlockSpec((128, 3 * d9), lambda i: (0, 0)),
            pl.BlockSpec((ROWS, 3 * d9), lambda i: (0, 0)),
        ],
        out_specs=pl.BlockSpec((ROWS, d_model), lambda i: (i, 0)),
        out_shape=jax.ShapeDtypeStruct((N, d_model), jnp.float32),
    )(x2, feat0, W1e, W2, pos2)
    return out.reshape(B, S, d_model)


# speedup vs baseline: 1.0086x; 1.0086x over previous
import jax
import jax.numpy as jnp
from jax.experimental import pallas as pl

ROWS = 4096


def _fused_kernel(x_ref, feat_ref, oh_ref, w1_ref, w2_ref, out_ref):
    x = x_ref[:]
    n_cont = x.shape[1] - 3
    r = x.shape[0]
    d6 = w1_ref.shape[1]
    feat_ref[:, :n_cont] = x[:, :n_cont].astype(jnp.bfloat16)
    idx = x[:, n_cont:].astype(jnp.int32)
    j = jax.lax.broadcasted_iota(jnp.int32, (r, 128), 1)
    oh = (j == idx[:, 1:2]) | (j == idx[:, 0:1] + 15) | (j == idx[:, 2:3] + 79)
    oh_ref[:, :128] = oh.astype(jnp.bfloat16)
    out_ref[:, :d6] = jnp.dot(feat_ref[:], w1_ref[:],
                              preferred_element_type=jnp.float32)
    out_ref[:, d6:] = jnp.dot(oh_ref[:], w2_ref[:],
                              preferred_element_type=jnp.float32)


def kernel(x, W, b, table_dd, table_plate, table_mag, table_pos):
    B, S, F = x.shape
    n_cont = F - 3
    d6 = W.shape[1]
    d9 = table_dd.shape[1]
    d_model = d6 + 3 * d9
    N = B * S

    n_dd = table_dd.shape[0]
    n_plate = table_plate.shape[0]
    n_mag = table_mag.shape[0]

    W1e = jnp.zeros((128, d6), jnp.float32)
    W1e = W1e.at[:n_cont, :].set(W)
    W1e = W1e.at[n_cont:n_cont + S, :].set(table_pos[:S, :d6])
    W1e = W1e.at[n_cont + S, :].set(b)
    W1e = W1e.astype(jnp.bfloat16)

    W2 = jnp.zeros((256, 3 * d9), jnp.float32)
    W2 = W2.at[:n_dd, :d9].set(table_dd)
    W2 = W2.at[n_dd:n_dd + n_plate, d9:2 * d9].set(table_plate)
    W2 = W2.at[n_dd + n_plate:n_dd + n_plate + n_mag, 2 * d9:].set(table_mag)
    W2 = W2.at[128:128 + S, :].set(table_pos[:S, d6:])
    W2 = W2.astype(jnp.bfloat16)

    rmod = jnp.arange(ROWS, dtype=jnp.int32) % S
    feat0 = jnp.zeros((ROWS, 128), jnp.bfloat16)
    feat0 = feat0.at[:, n_cont:n_cont + S].set(
        jax.nn.one_hot(rmod, S, dtype=jnp.bfloat16))
    feat0 = feat0.at[:, n_cont + S].set(jnp.bfloat16(1))

    oh0 = jnp.zeros((ROWS, 256), jnp.bfloat16)
    oh0 = oh0.at[:, 128:128 + S].set(jax.nn.one_hot(rmod, S,
                                                    dtype=jnp.bfloat16))

    x2 = x.reshape(N, F)
    out = pl.pallas_call(
        _fused_kernel,
        grid=(N // ROWS,),
        in_specs=[
            pl.BlockSpec((ROWS, F), lambda i: (i, 0)),
            pl.BlockSpec((ROWS, 128), lambda i: (0, 0)),
            pl.BlockSpec((ROWS, 256), lambda i: (0, 0)),
            pl.BlockSpec((128, d6), lambda i: (0, 0)),
            pl.BlockSpec((256, 3 * d9), lambda i: (0, 0)),
        ],
        out_specs=pl.BlockSpec((ROWS, d_model), lambda i: (i, 0)),
        out_shape=jax.ShapeDtypeStruct((N, d_model), jnp.float32),
    )(x2, feat0, oh0, W1e, W2)
    return out.reshape(B, S, d_model)
